# baseline (speedup 1.0000x reference)
import functools

import jax
import jax.numpy as jnp
from jax import lax
from jax.experimental import pallas as pl
from jax.experimental.pallas import tpu as pltpu
from jax.experimental.pallas import tpu_sc as plsc

L = 16
NC = 2
NS = 16
NW = NC * NS
C = 128
C2 = 64
BLK = 16
DEP = 4
SLOWC = 0


def _tc_prep(ns_ref, w_ref, at_ref, h_ref, s_ref, t_ref):
    h = jnp.dot(ns_ref[...], w_ref[...], preferred_element_type=jnp.float32)
    h_ref[...] = h
    st = lax.dot_general(
        at_ref[...], h, (((1,), (1,)), ((), ())),
        preferred_element_type=jnp.float32)
    s_ref[...] = st[0]
    t_ref[...] = st[1]


def _tc_finish(u_ref, as_ref, o_ref):
    u = u_ref[0] + u_ref[1]
    d = (as_ref[0] + as_ref[1])[:, None]
    o_ref[...] = jnp.where(d > 0.0, u / jnp.where(d > 0.0, d, 1.0), 0.0)


def _sc_body(h_hbm, s_hbm, t_hbm, dst_hbm, src_hbm, u_out, a_out,
             dst8, src8, att_b, sd_b, ts_b, hr,
             sem_h, sem_sd, sem_ts, sem_u, sem_a, sem_e, u_sh, as_sh,
             *, kj_split, np_):
    cid = lax.axis_index("c")
    sid = lax.axis_index("s")
    kjs, kjf = kj_split
    my_kj = jnp.where(cid == SLOWC, kjs, kjf)
    base = jnp.where(cid == SLOWC, sid * kjs, NS * kjs + sid * kjf)
    rows = np_ // NS
    z16 = jnp.zeros((L,), jnp.float32)

    def zw(i, c_):
        for k in range(8):
            hr[0, i, pl.ds(k * L, L)] = z16
        return c_
    lax.fori_loop(0, C2, zw, 0)
    for k in range(C2 // L):
        sd_b[0, pl.ds(k * L, L)] = z16

    def zu(i, c_):
        pltpu.sync_copy(hr.at[0], u_sh.at[pl.ds(sid * rows + i * C2, C2)])
        pltpu.sync_copy(sd_b.at[0], as_sh.at[pl.ds(sid * rows + i * C2, C2)])
        return c_
    lax.fori_loop(0, rows // C2, zu, 0)

    plsc.subcore_barrier()

    nblk = my_kj // BLK

    def gathers(blk_buf, j, b):
        d_sd = pltpu.async_copy(
            s_hbm.at[dst8.at[blk_buf, j]], sd_b.at[b], sem_sd.at[b])
        d_ts = pltpu.async_copy(
            t_hbm.at[src8.at[blk_buf, j]], ts_b.at[b], sem_ts.at[b])
        d_h = pltpu.async_copy(
            h_hbm.at[src8.at[blk_buf, j]], hr.at[b], sem_h.at[b])
        return d_sd, d_ts, d_h

    pltpu.sync_copy(dst_hbm.at[pl.ds(base, BLK)], dst8.at[0])
    pltpu.sync_copy(src_hbm.at[pl.ds(base, BLK)], src8.at[0])

    def body(blk, c_):
        cur = lax.rem(blk, 2)
        nxt = lax.rem(blk + 1, 2)
        nb = lax.min(blk + 1, nblk - 1) * BLK
        d_ed = pltpu.async_copy(
            dst_hbm.at[pl.ds(base + nb, BLK)], dst8.at[nxt], sem_e.at[0])
        d_es = pltpu.async_copy(
            src_hbm.at[pl.ds(base + nb, BLK)], src8.at[nxt], sem_e.at[1])

        pend = [None] * DEP
        d_g = [None] * DEP
        for j in range(DEP - 1):
            d_g[j] = gathers(cur, j, j)
        for j in range(BLK):
            b = j % DEP
            jn = j + DEP - 1
            if jn < BLK:
                bn = jn % DEP
                if pend[bn] is not None:
                    pend[bn][0].wait()
                    pend[bn][1].wait()
                    pend[bn] = None
                d_g[bn] = gathers(cur, jn, bn)
            d_sd, d_ts, d_h = d_g[b]
            d_sd.wait()
            d_ts.wait()

            def att_k(k, c3, b=b):
                x = sd_b[b, pl.ds(k * L, L)] + ts_b[b, pl.ds(k * L, L)]
                x = jnp.maximum(x, 0.2 * x)
                x = jnp.minimum(jnp.maximum(x, -2.0), 2.0)
                att_b[b, pl.ds(k * L, L)] = jnp.exp(x)
                return c3
            lax.fori_loop(0, C2 // L, att_k, 0)
            d_sa = pltpu.async_copy(
                att_b.at[b], as_sh.at[dst8.at[cur, j]], sem_a.at[b],
                add=True)

            d_h.wait()

            def wbody(j16, c3, b=b):
                attv = att_b[b, pl.ds(j16 * L, L)]
                for i16 in range(L):
                    a = attv[i16]
                    i = j16 * L + i16
                    for k in range(8):
                        hr[b, i, pl.ds(k * L, L)] = (
                            hr[b, i, pl.ds(k * L, L)] * a)
                return c3
            lax.fori_loop(0, C2 // L, wbody, 0)
            d_su = pltpu.async_copy(
                hr.at[b], u_sh.at[dst8.at[cur, j]], sem_u.at[b], add=True)
            pend[b] = (d_su, d_sa)
        for b in range(DEP):
            if pend[b] is not None:
                pend[b][0].wait()
                pend[b][1].wait()
        d_ed.wait()
        d_es.wait()
        return c_
    lax.fori_loop(0, nblk, body, 0)

    plsc.subcore_barrier()

    pltpu.sync_copy(u_sh.at[pl.ds(sid * rows, rows)],
                    u_out.at[cid, pl.ds(sid * rows, rows)])
    pltpu.sync_copy(as_sh.at[pl.ds(sid * rows, rows)],
                    a_out.at[cid, pl.ds(sid * rows, rows)])


def kernel(node_states, edges, kernel, kernel_attention):
    n, d = node_states.shape
    u = kernel.shape[1]
    e = edges.shape[0]

    edges = edges.astype(jnp.int32)
    dst = edges[:, 0]
    src = edges[:, 1]

    rb = 512
    np_ = ((n + 1 + rb - 1) // rb) * rb
    kj = ((-(-e // (NW * C2)) + BLK - 1) // BLK) * BLK
    ep = NW * kj * C2
    kjs = max(BLK, ((2 * kj) // 4 // BLK) * BLK)
    kjf = 2 * kj - kjs

    ns_p = jnp.pad(node_states, ((0, np_ - n), (0, 0)))
    at = kernel_attention.reshape(2, u)
    dst_p = jnp.concatenate(
        [dst, jnp.full((ep - e,), n, jnp.int32)]).reshape(NW * kj, C2)
    src_p = jnp.concatenate(
        [src, jnp.zeros((ep - e,), jnp.int32)]).reshape(NW * kj, C2)

    h, s, t = pl.pallas_call(
        _tc_prep,
        grid=(np_ // rb,),
        in_specs=[
            pl.BlockSpec((rb, d), lambda i: (i, 0)),
            pl.BlockSpec((d, u), lambda i: (0, 0)),
            pl.BlockSpec((2, u), lambda i: (0, 0)),
        ],
        out_specs=[
            pl.BlockSpec((rb, u), lambda i: (i, 0)),
            pl.BlockSpec((rb,), lambda i: (i,)),
            pl.BlockSpec((rb,), lambda i: (i,)),
        ],
        out_shape=[
            jax.ShapeDtypeStruct((np_, u), jnp.float32),
            jax.ShapeDtypeStruct((np_,), jnp.float32),
            jax.ShapeDtypeStruct((np_,), jnp.float32),
        ],
    )(ns_p, kernel, at)

    mesh = plsc.VectorSubcoreMesh(core_axis_name="c", subcore_axis_name="s")
    u_part, a_part = pl.kernel(
        functools.partial(_sc_body, kj_split=(kjs, kjf), np_=np_),
        out_type=[
            jax.ShapeDtypeStruct((NC, np_, u), jnp.float32),
            jax.ShapeDtypeStruct((NC, np_), jnp.float32),
        ],
        mesh=mesh,
        compiler_params=pltpu.CompilerParams(needs_layout_passes=False),
        scratch_types=[
            pltpu.VMEM((2, BLK, C2), jnp.int32),
            pltpu.VMEM((2, BLK, C2), jnp.int32),
            pltpu.VMEM((DEP, C2), jnp.float32),
            pltpu.VMEM((DEP, C2), jnp.float32),
            pltpu.VMEM((DEP, C2), jnp.float32),
            pltpu.VMEM((DEP, C2, u), jnp.float32),
            pltpu.SemaphoreType.DMA((DEP,)),
            pltpu.SemaphoreType.DMA((DEP,)),
            pltpu.SemaphoreType.DMA((DEP,)),
            pltpu.SemaphoreType.DMA((DEP,)),
            pltpu.SemaphoreType.DMA((DEP,)),
            pltpu.SemaphoreType.DMA((2,)),
            pltpu.VMEM_SHARED((np_, u), jnp.float32),
            pltpu.VMEM_SHARED((np_,), jnp.float32),
        ],
    )(h, s, t, dst_p, src_p)

    out = pl.pallas_call(
        _tc_finish,
        grid=(np_ // rb,),
        in_specs=[
            pl.BlockSpec((NC, rb, u), lambda i: (0, i, 0)),
            pl.BlockSpec((NC, rb), lambda i: (0, i)),
        ],
        out_specs=pl.BlockSpec((rb, u), lambda i: (i, 0)),
        out_shape=jax.ShapeDtypeStruct((np_, u), jnp.float32),
    )(u_part, a_part)

    return out[:n]


# speedup vs baseline: 1.0511x; 1.0511x over previous
import functools

import jax
import jax.numpy as jnp
from jax import lax
from jax.experimental import pallas as pl
from jax.experimental.pallas import tpu as pltpu
from jax.experimental.pallas import tpu_sc as plsc

L = 16
NC = 2
NS = 16
NW = NC * NS
C = 128
C2 = 64
BLK = 16
DEP = 4
SLOWC = 1


def _tc_prep(ns_ref, w_ref, at_ref, h_ref, s_ref, t_ref):
    h = jnp.dot(ns_ref[...], w_ref[...], preferred_element_type=jnp.float32)
    h_ref[...] = h
    st = lax.dot_general(
        at_ref[...], h, (((1,), (1,)), ((), ())),
        preferred_element_type=jnp.float32)
    s_ref[...] = st[0]
    t_ref[...] = st[1]


def _tc_finish(u_ref, as_ref, o_ref):
    u = u_ref[0] + u_ref[1]
    d = (as_ref[0] + as_ref[1])[:, None]
    o_ref[...] = jnp.where(d > 0.0, u / jnp.where(d > 0.0, d, 1.0), 0.0)


def _sc_body(h_hbm, s_hbm, t_hbm, dst_hbm, src_hbm, u_out, a_out,
             dst8, src8, att_b, sd_b, ts_b, hr,
             sem_h, sem_sd, sem_ts, sem_u, sem_a, sem_e, u_sh, as_sh,
             *, kj_split, np_):
    cid = lax.axis_index("c")
    sid = lax.axis_index("s")
    kjs, kjf = kj_split
    my_kj = jnp.where(cid == SLOWC, kjs, kjf)
    base = jnp.where(cid == SLOWC, sid * kjs, NS * kjs + sid * kjf)
    rows = np_ // NS
    z16 = jnp.zeros((L,), jnp.float32)

    def zw(i, c_):
        for k in range(8):
            hr[0, i, pl.ds(k * L, L)] = z16
        return c_
    lax.fori_loop(0, C2, zw, 0)
    for k in range(C2 // L):
        sd_b[0, pl.ds(k * L, L)] = z16

    def zu(i, c_):
        pltpu.sync_copy(hr.at[0], u_sh.at[pl.ds(sid * rows + i * C2, C2)])
        pltpu.sync_copy(sd_b.at[0], as_sh.at[pl.ds(sid * rows + i * C2, C2)])
        return c_
    lax.fori_loop(0, rows // C2, zu, 0)

    plsc.subcore_barrier()

    nblk = my_kj // BLK

    def gathers(blk_buf, j, b):
        d_sd = pltpu.async_copy(
            s_hbm.at[dst8.at[blk_buf, j]], sd_b.at[b], sem_sd.at[b])
        d_ts = pltpu.async_copy(
            t_hbm.at[src8.at[blk_buf, j]], ts_b.at[b], sem_ts.at[b])
        d_h = pltpu.async_copy(
            h_hbm.at[src8.at[blk_buf, j]], hr.at[b], sem_h.at[b])
        return d_sd, d_ts, d_h

    pltpu.sync_copy(dst_hbm.at[pl.ds(base, BLK)], dst8.at[0])
    pltpu.sync_copy(src_hbm.at[pl.ds(base, BLK)], src8.at[0])

    def body(blk, c_):
        cur = lax.rem(blk, 2)
        nxt = lax.rem(blk + 1, 2)
        nb = lax.min(blk + 1, nblk - 1) * BLK
        d_ed = pltpu.async_copy(
            dst_hbm.at[pl.ds(base + nb, BLK)], dst8.at[nxt], sem_e.at[0])
        d_es = pltpu.async_copy(
            src_hbm.at[pl.ds(base + nb, BLK)], src8.at[nxt], sem_e.at[1])

        pend = [None] * DEP
        d_g = [None] * DEP
        for j in range(DEP - 1):
            d_g[j] = gathers(cur, j, j)
        for j in range(BLK):
            b = j % DEP
            jn = j + DEP - 1
            if jn < BLK:
                bn = jn % DEP
                if pend[bn] is not None:
                    pend[bn][0].wait()
                    pend[bn][1].wait()
                    pend[bn] = None
                d_g[bn] = gathers(cur, jn, bn)
            d_sd, d_ts, d_h = d_g[b]
            d_sd.wait()
            d_ts.wait()

            def att_k(k, c3, b=b):
                x = sd_b[b, pl.ds(k * L, L)] + ts_b[b, pl.ds(k * L, L)]
                x = jnp.maximum(x, 0.2 * x)
                x = jnp.minimum(jnp.maximum(x, -2.0), 2.0)
                att_b[b, pl.ds(k * L, L)] = jnp.exp(x)
                return c3
            lax.fori_loop(0, C2 // L, att_k, 0)
            d_sa = pltpu.async_copy(
                att_b.at[b], as_sh.at[dst8.at[cur, j]], sem_a.at[b],
                add=True)

            d_h.wait()

            def wbody(j16, c3, b=b):
                attv = att_b[b, pl.ds(j16 * L, L)]
                for i16 in range(L):
                    a = attv[i16]
                    i = j16 * L + i16
                    for k in range(8):
                        hr[b, i, pl.ds(k * L, L)] = (
                            hr[b, i, pl.ds(k * L, L)] * a)
                return c3
            lax.fori_loop(0, C2 // L, wbody, 0)
            d_su = pltpu.async_copy(
                hr.at[b], u_sh.at[dst8.at[cur, j]], sem_u.at[b], add=True)
            pend[b] = (d_su, d_sa)
        for b in range(DEP):
            if pend[b] is not None:
                pend[b][0].wait()
                pend[b][1].wait()
        d_ed.wait()
        d_es.wait()
        return c_
    lax.fori_loop(0, nblk, body, 0)

    plsc.subcore_barrier()

    pltpu.sync_copy(u_sh.at[pl.ds(sid * rows, rows)],
                    u_out.at[cid, pl.ds(sid * rows, rows)])
    pltpu.sync_copy(as_sh.at[pl.ds(sid * rows, rows)],
                    a_out.at[cid, pl.ds(sid * rows, rows)])


def kernel(node_states, edges, kernel, kernel_attention):
    n, d = node_states.shape
    u = kernel.shape[1]
    e = edges.shape[0]

    edges = edges.astype(jnp.int32)
    dst = edges[:, 0]
    src = edges[:, 1]

    rb = 512
    np_ = ((n + 1 + rb - 1) // rb) * rb
    kj = ((-(-e // (NW * C2)) + BLK - 1) // BLK) * BLK
    ep = NW * kj * C2
    kjs = max(BLK, ((2 * kj) // 4 // BLK) * BLK)
    kjf = 2 * kj - kjs

    ns_p = jnp.pad(node_states, ((0, np_ - n), (0, 0)))
    at = kernel_attention.reshape(2, u)
    dst_p = jnp.concatenate(
        [dst, jnp.full((ep - e,), n, jnp.int32)]).reshape(NW * kj, C2)
    src_p = jnp.concatenate(
        [src, jnp.zeros((ep - e,), jnp.int32)]).reshape(NW * kj, C2)

    h, s, t = pl.pallas_call(
        _tc_prep,
        grid=(np_ // rb,),
        in_specs=[
            pl.BlockSpec((rb, d), lambda i: (i, 0)),
            pl.BlockSpec((d, u), lambda i: (0, 0)),
            pl.BlockSpec((2, u), lambda i: (0, 0)),
        ],
        out_specs=[
            pl.BlockSpec((rb, u), lambda i: (i, 0)),
            pl.BlockSpec((rb,), lambda i: (i,)),
            pl.BlockSpec((rb,), lambda i: (i,)),
        ],
        out_shape=[
            jax.ShapeDtypeStruct((np_, u), jnp.float32),
            jax.ShapeDtypeStruct((np_,), jnp.float32),
            jax.ShapeDtypeStruct((np_,), jnp.float32),
        ],
    )(ns_p, kernel, at)

    mesh = plsc.VectorSubcoreMesh(core_axis_name="c", subcore_axis_name="s")
    u_part, a_part = pl.kernel(
        functools.partial(_sc_body, kj_split=(kjs, kjf), np_=np_),
        out_type=[
            jax.ShapeDtypeStruct((NC, np_, u), jnp.float32),
            jax.ShapeDtypeStruct((NC, np_), jnp.float32),
        ],
        mesh=mesh,
        compiler_params=pltpu.CompilerParams(needs_layout_passes=False),
        scratch_types=[
            pltpu.VMEM((2, BLK, C2), jnp.int32),
            pltpu.VMEM((2, BLK, C2), jnp.int32),
            pltpu.VMEM((DEP, C2), jnp.float32),
            pltpu.VMEM((DEP, C2), jnp.float32),
            pltpu.VMEM((DEP, C2), jnp.float32),
            pltpu.VMEM((DEP, C2, u), jnp.float32),
            pltpu.SemaphoreType.DMA((DEP,)),
            pltpu.SemaphoreType.DMA((DEP,)),
            pltpu.SemaphoreType.DMA((DEP,)),
            pltpu.SemaphoreType.DMA((DEP,)),
            pltpu.SemaphoreType.DMA((DEP,)),
            pltpu.SemaphoreType.DMA((2,)),
            pltpu.VMEM_SHARED((np_, u), jnp.float32),
            pltpu.VMEM_SHARED((np_,), jnp.float32),
        ],
    )(h, s, t, dst_p, src_p)

    out = pl.pallas_call(
        _tc_finish,
        grid=(np_ // rb,),
        in_specs=[
            pl.BlockSpec((NC, rb, u), lambda i: (0, i, 0)),
            pl.BlockSpec((NC, rb), lambda i: (0, i)),
        ],
        out_specs=pl.BlockSpec((rb, u), lambda i: (i, 0)),
        out_shape=jax.ShapeDtypeStruct((np_, u), jnp.float32),
    )(u_part, a_part)

    return out[:n]


# speedup vs baseline: 1.1783x; 1.1211x over previous
import functools

import jax
import jax.numpy as jnp
from jax import lax
from jax.experimental import pallas as pl
from jax.experimental.pallas import tpu as pltpu
from jax.experimental.pallas import tpu_sc as plsc

L = 16
NC = 2
NS = 16
NW = NC * NS
C = 128
C2 = 64
BLK = 16
DEP = 4
SLOWC = 1


def _tc_prep(ns_ref, w_ref, at_ref, h_ref, s_ref, t_ref):
    h = jnp.dot(ns_ref[...], w_ref[...], preferred_element_type=jnp.float32)
    h_ref[...] = h
    st = lax.dot_general(
        at_ref[...], h, (((1,), (1,)), ((), ())),
        preferred_element_type=jnp.float32)
    s_ref[...] = st[0]
    t_ref[...] = st[1]


def _tc_finish(u_ref, as_ref, o_ref):
    u = u_ref[0] + u_ref[1]
    d = (as_ref[0] + as_ref[1])[:, None]
    o_ref[...] = jnp.where(d > 0.0, u / jnp.where(d > 0.0, d, 1.0), 0.0)


def _sc_body(h_hbm, s_hbm, t_hbm, dst_hbm, src_hbm, u_out, a_out,
             dst8, src8, att_b, sd_b, ts_b, hr,
             sem_h, sem_sd, sem_ts, sem_u, sem_a, sem_e, u_sh, as_sh,
             s_sh, t_sh, *, kj_split, np_):
    cid = lax.axis_index("c")
    sid = lax.axis_index("s")
    kjs, kjf = kj_split
    my_kj = jnp.where(cid == SLOWC, kjs, kjf)
    base = jnp.where(cid == SLOWC, sid * kjs, NS * kjs + sid * kjf)
    rows = np_ // NS
    z16 = jnp.zeros((L,), jnp.float32)

    def zw(i, c_):
        for k in range(8):
            hr[0, i, pl.ds(k * L, L)] = z16
        return c_
    lax.fori_loop(0, C2, zw, 0)
    for k in range(C2 // L):
        sd_b[0, pl.ds(k * L, L)] = z16

    def zu(i, c_):
        pltpu.sync_copy(hr.at[0], u_sh.at[pl.ds(sid * rows + i * C2, C2)])
        pltpu.sync_copy(sd_b.at[0], as_sh.at[pl.ds(sid * rows + i * C2, C2)])
        return c_
    lax.fori_loop(0, rows // C2, zu, 0)
    pltpu.sync_copy(s_hbm.at[pl.ds(sid * rows, rows)],
                    s_sh.at[pl.ds(sid * rows, rows)])
    pltpu.sync_copy(t_hbm.at[pl.ds(sid * rows, rows)],
                    t_sh.at[pl.ds(sid * rows, rows)])

    plsc.subcore_barrier()

    nblk = my_kj // BLK

    def gathers(blk_buf, j, b):
        d_sd = pltpu.async_copy(
            s_sh.at[dst8.at[blk_buf, j]], sd_b.at[b], sem_sd.at[b])
        d_ts = pltpu.async_copy(
            t_sh.at[src8.at[blk_buf, j]], ts_b.at[b], sem_ts.at[b])
        d_h = pltpu.async_copy(
            h_hbm.at[src8.at[blk_buf, j]], hr.at[b], sem_h.at[b])
        return d_sd, d_ts, d_h

    pltpu.sync_copy(dst_hbm.at[pl.ds(base, BLK)], dst8.at[0])
    pltpu.sync_copy(src_hbm.at[pl.ds(base, BLK)], src8.at[0])

    def body(blk, c_):
        cur = lax.rem(blk, 2)
        nxt = lax.rem(blk + 1, 2)
        nb = lax.min(blk + 1, nblk - 1) * BLK
        d_ed = pltpu.async_copy(
            dst_hbm.at[pl.ds(base + nb, BLK)], dst8.at[nxt], sem_e.at[0])
        d_es = pltpu.async_copy(
            src_hbm.at[pl.ds(base + nb, BLK)], src8.at[nxt], sem_e.at[1])

        pend = [None] * DEP
        d_g = [None] * DEP
        for j in range(DEP - 1):
            d_g[j] = gathers(cur, j, j)
        for j in range(BLK):
            b = j % DEP
            jn = j + DEP - 1
            if jn < BLK:
                bn = jn % DEP
                if pend[bn] is not None:
                    pend[bn][0].wait()
                    pend[bn][1].wait()
                    pend[bn] = None
                d_g[bn] = gathers(cur, jn, bn)
            d_sd, d_ts, d_h = d_g[b]
            d_sd.wait()
            d_ts.wait()

            def att_k(k, c3, b=b):
                x = sd_b[b, pl.ds(k * L, L)] + ts_b[b, pl.ds(k * L, L)]
                x = jnp.maximum(x, 0.2 * x)
                x = jnp.minimum(jnp.maximum(x, -2.0), 2.0)
                att_b[b, pl.ds(k * L, L)] = jnp.exp(x)
                return c3
            lax.fori_loop(0, C2 // L, att_k, 0)
            d_sa = pltpu.async_copy(
                att_b.at[b], as_sh.at[dst8.at[cur, j]], sem_a.at[b],
                add=True)

            d_h.wait()

            def wbody(j16, c3, b=b):
                attv = att_b[b, pl.ds(j16 * L, L)]
                for i16 in range(L):
                    a = attv[i16]
                    i = j16 * L + i16
                    for k in range(8):
                        hr[b, i, pl.ds(k * L, L)] = (
                            hr[b, i, pl.ds(k * L, L)] * a)
                return c3
            lax.fori_loop(0, C2 // L, wbody, 0)
            d_su = pltpu.async_copy(
                hr.at[b], u_sh.at[dst8.at[cur, j]], sem_u.at[b], add=True)
            pend[b] = (d_su, d_sa)
        for b in range(DEP):
            if pend[b] is not None:
                pend[b][0].wait()
                pend[b][1].wait()
        d_ed.wait()
        d_es.wait()
        return c_
    lax.fori_loop(0, nblk, body, 0)

    plsc.subcore_barrier()

    pltpu.sync_copy(u_sh.at[pl.ds(sid * rows, rows)],
                    u_out.at[cid, pl.ds(sid * rows, rows)])
    pltpu.sync_copy(as_sh.at[pl.ds(sid * rows, rows)],
                    a_out.at[cid, pl.ds(sid * rows, rows)])


def kernel(node_states, edges, kernel, kernel_attention):
    n, d = node_states.shape
    u = kernel.shape[1]
    e = edges.shape[0]

    edges = edges.astype(jnp.int32)
    dst = edges[:, 0]
    src = edges[:, 1]

    rb = 512
    np_ = ((n + 1 + rb - 1) // rb) * rb
    kj = ((-(-e // (NW * C2)) + BLK - 1) // BLK) * BLK
    ep = NW * kj * C2
    kjs = kj
    kjf = kj

    ns_p = jnp.pad(node_states, ((0, np_ - n), (0, 0)))
    at = kernel_attention.reshape(2, u)
    dst_p = jnp.concatenate(
        [dst, jnp.full((ep - e,), n, jnp.int32)]).reshape(NW * kj, C2)
    src_p = jnp.concatenate(
        [src, jnp.zeros((ep - e,), jnp.int32)]).reshape(NW * kj, C2)

    h, s, t = pl.pallas_call(
        _tc_prep,
        grid=(np_ // rb,),
        in_specs=[
            pl.BlockSpec((rb, d), lambda i: (i, 0)),
            pl.BlockSpec((d, u), lambda i: (0, 0)),
            pl.BlockSpec((2, u), lambda i: (0, 0)),
        ],
        out_specs=[
            pl.BlockSpec((rb, u), lambda i: (i, 0)),
            pl.BlockSpec((rb,), lambda i: (i,)),
            pl.BlockSpec((rb,), lambda i: (i,)),
        ],
        out_shape=[
            jax.ShapeDtypeStruct((np_, u), jnp.float32),
            jax.ShapeDtypeStruct((np_,), jnp.float32),
            jax.ShapeDtypeStruct((np_,), jnp.float32),
        ],
    )(ns_p, kernel, at)

    mesh = plsc.VectorSubcoreMesh(core_axis_name="c", subcore_axis_name="s")
    u_part, a_part = pl.kernel(
        functools.partial(_sc_body, kj_split=(kjs, kjf), np_=np_),
        out_type=[
            jax.ShapeDtypeStruct((NC, np_, u), jnp.float32),
            jax.ShapeDtypeStruct((NC, np_), jnp.float32),
        ],
        mesh=mesh,
        compiler_params=pltpu.CompilerParams(needs_layout_passes=False),
        scratch_types=[
            pltpu.VMEM((2, BLK, C2), jnp.int32),
            pltpu.VMEM((2, BLK, C2), jnp.int32),
            pltpu.VMEM((DEP, C2), jnp.float32),
            pltpu.VMEM((DEP, C2), jnp.float32),
            pltpu.VMEM((DEP, C2), jnp.float32),
            pltpu.VMEM((DEP, C2, u), jnp.float32),
            pltpu.SemaphoreType.DMA((DEP,)),
            pltpu.SemaphoreType.DMA((DEP,)),
            pltpu.SemaphoreType.DMA((DEP,)),
            pltpu.SemaphoreType.DMA((DEP,)),
            pltpu.SemaphoreType.DMA((DEP,)),
            pltpu.SemaphoreType.DMA((2,)),
            pltpu.VMEM_SHARED((np_, u), jnp.float32),
            pltpu.VMEM_SHARED((np_,), jnp.float32),
            pltpu.VMEM_SHARED((np_,), jnp.float32),
            pltpu.VMEM_SHARED((np_,), jnp.float32),
        ],
    )(h, s, t, dst_p, src_p)

    out = pl.pallas_call(
        _tc_finish,
        grid=(np_ // rb,),
        in_specs=[
            pl.BlockSpec((NC, rb, u), lambda i: (0, i, 0)),
            pl.BlockSpec((NC, rb), lambda i: (0, i)),
        ],
        out_specs=pl.BlockSpec((rb, u), lambda i: (i, 0)),
        out_shape=jax.ShapeDtypeStruct((np_, u), jnp.float32),
    )(u_part, a_part)

    return out[:n]


# speedup vs baseline: 1.3895x; 1.1792x over previous
import functools

import jax
import jax.numpy as jnp
from jax import lax
from jax.experimental import pallas as pl
from jax.experimental.pallas import tpu as pltpu
from jax.experimental.pallas import tpu_sc as plsc

L = 16
NC = 2
NS = 16
NW = NC * NS
C = 128
C2 = 64
BLK = 16
DEP = 3
SLOWC = 1


def _tc_prep(ns_ref, w_ref, at_ref, h_ref, s_ref, t_ref):
    h = jnp.dot(ns_ref[...], w_ref[...], preferred_element_type=jnp.float32)
    h_ref[...] = h.astype(jnp.bfloat16)
    st = lax.dot_general(
        at_ref[...], h, (((1,), (1,)), ((), ())),
        preferred_element_type=jnp.float32)
    s_ref[...] = st[0]
    t_ref[...] = st[1]


def _tc_finish(u_ref, as_ref, o_ref):
    u = u_ref[0] + u_ref[1]
    d = (as_ref[0] + as_ref[1])[:, None]
    o_ref[...] = jnp.where(d > 0.0, u / jnp.where(d > 0.0, d, 1.0), 0.0)


def _sc_body(h_hbm, s_hbm, t_hbm, dst_hbm, src_hbm, u_out, a_out,
             dst8, src8, att_b, sd_b, ts_b, hr, wrows,
             sem_h, sem_sd, sem_ts, sem_u, sem_a, sem_e, u_sh, as_sh,
             s_sh, t_sh, *, kj_split, np_):
    cid = lax.axis_index("c")
    sid = lax.axis_index("s")
    kjs, kjf = kj_split
    my_kj = jnp.where(cid == SLOWC, kjs, kjf)
    base = jnp.where(cid == SLOWC, sid * kjs, NS * kjs + sid * kjf)
    rows = np_ // NS
    z16 = jnp.zeros((L,), jnp.float32)

    def zw(i, c_):
        for k in range(8):
            wrows[0, i, pl.ds(k * L, L)] = z16
        return c_
    lax.fori_loop(0, C2, zw, 0)
    for k in range(C2 // L):
        sd_b[0, pl.ds(k * L, L)] = z16

    def zu(i, c_):
        pltpu.sync_copy(wrows.at[0], u_sh.at[pl.ds(sid * rows + i * C2, C2)])
        pltpu.sync_copy(sd_b.at[0], as_sh.at[pl.ds(sid * rows + i * C2, C2)])
        return c_
    lax.fori_loop(0, rows // C2, zu, 0)
    pltpu.sync_copy(s_hbm.at[pl.ds(sid * rows, rows)],
                    s_sh.at[pl.ds(sid * rows, rows)])
    pltpu.sync_copy(t_hbm.at[pl.ds(sid * rows, rows)],
                    t_sh.at[pl.ds(sid * rows, rows)])

    plsc.subcore_barrier()

    nblk = my_kj // BLK

    def gathers(blk_buf, j, b):
        d_sd = pltpu.async_copy(
            s_sh.at[dst8.at[blk_buf, j]], sd_b.at[b], sem_sd.at[b])
        d_ts = pltpu.async_copy(
            t_sh.at[src8.at[blk_buf, j]], ts_b.at[b], sem_ts.at[b])
        d_h = pltpu.async_copy(
            h_hbm.at[src8.at[blk_buf, j]], hr.at[b], sem_h.at[b])
        return d_sd, d_ts, d_h

    pltpu.sync_copy(dst_hbm.at[pl.ds(base, BLK)], dst8.at[0])
    pltpu.sync_copy(src_hbm.at[pl.ds(base, BLK)], src8.at[0])

    def body(blk, c_):
        cur = lax.rem(blk, 2)
        nxt = lax.rem(blk + 1, 2)
        nb = lax.min(blk + 1, nblk - 1) * BLK
        d_ed = pltpu.async_copy(
            dst_hbm.at[pl.ds(base + nb, BLK)], dst8.at[nxt], sem_e.at[0])
        d_es = pltpu.async_copy(
            src_hbm.at[pl.ds(base + nb, BLK)], src8.at[nxt], sem_e.at[1])

        pend = [None] * DEP
        d_g = [None] * DEP
        for j in range(DEP - 1):
            d_g[j] = gathers(cur, j, j)
        for j in range(BLK):
            b = j % DEP
            jn = j + DEP - 1
            if jn < BLK:
                bn = jn % DEP
                if pend[bn] is not None:
                    pend[bn][0].wait()
                    pend[bn][1].wait()
                    pend[bn] = None
                d_g[bn] = gathers(cur, jn, bn)
            d_sd, d_ts, d_h = d_g[b]
            d_sd.wait()
            d_ts.wait()

            def att_k(k, c3, b=b):
                x = sd_b[b, pl.ds(k * L, L)] + ts_b[b, pl.ds(k * L, L)]
                x = jnp.maximum(x, 0.2 * x)
                x = jnp.minimum(jnp.maximum(x, -2.0), 2.0)
                att_b[b, pl.ds(k * L, L)] = jnp.exp(x)
                return c3
            lax.fori_loop(0, C2 // L, att_k, 0)
            d_sa = pltpu.async_copy(
                att_b.at[b], as_sh.at[dst8.at[cur, j]], sem_a.at[b],
                add=True)

            d_h.wait()

            def wbody(j16, c3, b=b):
                attv = att_b[b, pl.ds(j16 * L, L)]
                for i16 in range(L):
                    a = attv[i16]
                    i = j16 * L + i16
                    for kk in range(4):
                        x = hr[b, i, pl.ds(kk * L, L)]
                        ev = plsc.bitcast(lax.shift_left(x, 16),
                                          jnp.float32)
                        od = plsc.bitcast(
                            lax.bitwise_and(x, jnp.int32(-65536)),
                            jnp.float32)
                        wrows[b, i, pl.ds(kk * 32, L)] = ev * a
                        wrows[b, i, pl.ds(kk * 32 + L, L)] = od * a
                return c3
            lax.fori_loop(0, C2 // L, wbody, 0)
            d_su = pltpu.async_copy(
                wrows.at[b], u_sh.at[dst8.at[cur, j]], sem_u.at[b],
                add=True)
            pend[b] = (d_su, d_sa)
        for b in range(DEP):
            if pend[b] is not None:
                pend[b][0].wait()
                pend[b][1].wait()
        d_ed.wait()
        d_es.wait()
        return c_
    lax.fori_loop(0, nblk, body, 0)

    plsc.subcore_barrier()

    pltpu.sync_copy(u_sh.at[pl.ds(sid * rows, rows)],
                    u_out.at[cid, pl.ds(sid * rows, rows)])
    pltpu.sync_copy(as_sh.at[pl.ds(sid * rows, rows)],
                    a_out.at[cid, pl.ds(sid * rows, rows)])


def kernel(node_states, edges, kernel, kernel_attention):
    n, d = node_states.shape
    u = kernel.shape[1]
    e = edges.shape[0]

    edges = edges.astype(jnp.int32)
    dst = edges[:, 0]
    src = edges[:, 1]

    rb = 512
    np_ = ((n + 1 + rb - 1) // rb) * rb
    kj = ((-(-e // (NW * C2)) + BLK - 1) // BLK) * BLK
    ep = NW * kj * C2
    kjs = kj
    kjf = kj

    ns_p = jnp.pad(node_states, ((0, np_ - n), (0, 0)))
    perm = []
    for blk32 in range(0, u, 32):
        for m in range(16):
            perm.extend([blk32 + m, blk32 + 16 + m])
    perm = jnp.array(perm, jnp.int32)
    at = kernel_attention.reshape(2, u)[:, perm]
    kernel = kernel[:, perm]
    dst_p = jnp.concatenate(
        [dst, jnp.full((ep - e,), n, jnp.int32)]).reshape(NW * kj, C2)
    src_p = jnp.concatenate(
        [src, jnp.zeros((ep - e,), jnp.int32)]).reshape(NW * kj, C2)

    h, s, t = pl.pallas_call(
        _tc_prep,
        grid=(np_ // rb,),
        in_specs=[
            pl.BlockSpec((rb, d), lambda i: (i, 0)),
            pl.BlockSpec((d, u), lambda i: (0, 0)),
            pl.BlockSpec((2, u), lambda i: (0, 0)),
        ],
        out_specs=[
            pl.BlockSpec((rb, u), lambda i: (i, 0)),
            pl.BlockSpec((rb,), lambda i: (i,)),
            pl.BlockSpec((rb,), lambda i: (i,)),
        ],
        out_shape=[
            jax.ShapeDtypeStruct((np_, u), jnp.bfloat16),
            jax.ShapeDtypeStruct((np_,), jnp.float32),
            jax.ShapeDtypeStruct((np_,), jnp.float32),
        ],
    )(ns_p, kernel, at)
    h = lax.bitcast_convert_type(h.reshape(np_, u // 2, 2), jnp.int32)

    mesh = plsc.VectorSubcoreMesh(core_axis_name="c", subcore_axis_name="s")
    u_part, a_part = pl.kernel(
        functools.partial(_sc_body, kj_split=(kjs, kjf), np_=np_),
        out_type=[
            jax.ShapeDtypeStruct((NC, np_, u), jnp.float32),
            jax.ShapeDtypeStruct((NC, np_), jnp.float32),
        ],
        mesh=mesh,
        compiler_params=pltpu.CompilerParams(needs_layout_passes=False, use_tc_tiling_on_sc=False),
        scratch_types=[
            pltpu.VMEM((2, BLK, C2), jnp.int32),
            pltpu.VMEM((2, BLK, C2), jnp.int32),
            pltpu.VMEM((DEP, C2), jnp.float32),
            pltpu.VMEM((DEP, C2), jnp.float32),
            pltpu.VMEM((DEP, C2), jnp.float32),
            pltpu.VMEM((DEP, C2, u // 2), jnp.int32),
            pltpu.VMEM((DEP, C2, u), jnp.float32),
            pltpu.SemaphoreType.DMA((DEP,)),
            pltpu.SemaphoreType.DMA((DEP,)),
            pltpu.SemaphoreType.DMA((DEP,)),
            pltpu.SemaphoreType.DMA((DEP,)),
            pltpu.SemaphoreType.DMA((DEP,)),
            pltpu.SemaphoreType.DMA((2,)),
            pltpu.VMEM_SHARED((np_, u), jnp.float32),
            pltpu.VMEM_SHARED((np_,), jnp.float32),
            pltpu.VMEM_SHARED((np_,), jnp.float32),
            pltpu.VMEM_SHARED((np_,), jnp.float32),
        ],
    )(h, s, t, dst_p, src_p)

    out = pl.pallas_call(
        _tc_finish,
        grid=(np_ // rb,),
        in_specs=[
            pl.BlockSpec((NC, rb, u), lambda i: (0, i, 0)),
            pl.BlockSpec((NC, rb), lambda i: (0, i)),
        ],
        out_specs=pl.BlockSpec((rb, u), lambda i: (i, 0)),
        out_shape=jax.ShapeDtypeStruct((np_, u), jnp.float32),
    )(u_part, a_part)

    return out[:n]


# speedup vs baseline: 1.4738x; 1.0607x over previous
import functools

import jax
import jax.numpy as jnp
from jax import lax
from jax.experimental import pallas as pl
from jax.experimental.pallas import tpu as pltpu
from jax.experimental.pallas import tpu_sc as plsc

L = 16
NC = 2
NS = 16
NW = NC * NS
C = 128
C2 = 64
BLK = 16
DEP = 3
SLOWC = 1


def _tc_prep(ns_ref, w_ref, at_ref, h_ref, s_ref, t_ref):
    h = jnp.dot(ns_ref[...], w_ref[...], preferred_element_type=jnp.float32)
    h_ref[...] = h.astype(jnp.bfloat16)
    st = lax.dot_general(
        at_ref[...], h, (((1,), (1,)), ((), ())),
        preferred_element_type=jnp.float32)
    s_ref[...] = st[0]
    t_ref[...] = st[1]


def _tc_finish(u_ref, as_ref, o_ref):
    u = u_ref[0] + u_ref[1]
    d = (as_ref[0] + as_ref[1])[:, None]
    o_ref[...] = jnp.where(d > 0.0, u / jnp.where(d > 0.0, d, 1.0), 0.0)


def _sc_body(h_hbm, st_hbm, dst_hbm, src_hbm, u_out, a_out,
             dst8, src8, att_b, st_loc, hr, wrows,
             sem_h, sem_u, sem_a, sem_e, u_sh, as_sh,
             *, kj_split, np_):
    cid = lax.axis_index("c")
    sid = lax.axis_index("s")
    kjs, kjf = kj_split
    my_kj = jnp.where(cid == SLOWC, kjs, kjf)
    base = jnp.where(cid == SLOWC, sid * kjs, NS * kjs + sid * kjf)
    rows = np_ // NS
    z16 = jnp.zeros((L,), jnp.float32)

    def zw(i, c_):
        for k in range(8):
            wrows[0, i, pl.ds(k * L, L)] = z16
        return c_
    lax.fori_loop(0, C2, zw, 0)
    for k in range(C2 // L):
        att_b[0, pl.ds(k * L, L)] = z16

    def zu(i, c_):
        pltpu.sync_copy(wrows.at[0], u_sh.at[pl.ds(sid * rows + i * C2, C2)])
        pltpu.sync_copy(att_b.at[0], as_sh.at[pl.ds(sid * rows + i * C2, C2)])
        return c_
    lax.fori_loop(0, rows // C2, zu, 0)
    pltpu.sync_copy(st_hbm, st_loc)

    plsc.subcore_barrier()

    nblk = my_kj // BLK

    def gathers(blk_buf, j, b):
        return pltpu.async_copy(
            h_hbm.at[src8.at[blk_buf, j]], hr.at[b], sem_h.at[b])

    pltpu.sync_copy(dst_hbm.at[pl.ds(base, BLK)], dst8.at[0])
    pltpu.sync_copy(src_hbm.at[pl.ds(base, BLK)], src8.at[0])

    def body(blk, c_):
        cur = lax.rem(blk, 2)
        nxt = lax.rem(blk + 1, 2)
        nb = lax.min(blk + 1, nblk - 1) * BLK
        d_ed = pltpu.async_copy(
            dst_hbm.at[pl.ds(base + nb, BLK)], dst8.at[nxt], sem_e.at[0])
        d_es = pltpu.async_copy(
            src_hbm.at[pl.ds(base + nb, BLK)], src8.at[nxt], sem_e.at[1])

        pend = [None, None]
        d_g = [None] * DEP
        for j in range(DEP - 1):
            d_g[j] = gathers(cur, j, j)
        for j in range(BLK):
            b = j % DEP
            bw = j % 2
            jn = j + DEP - 1
            if jn < BLK:
                d_g[jn % DEP] = gathers(cur, jn, jn % DEP)
            if pend[bw] is not None:
                pend[bw][0].wait()
                pend[bw][1].wait()
                pend[bw] = None

            def att_k(k, c3, bw=bw, j=j):
                dv = dst8[cur, j, pl.ds(k * L, L)]
                sv = src8[cur, j, pl.ds(k * L, L)]
                ps = plsc.load_gather(st_loc, [dv])
                pt = plsc.load_gather(st_loc, [sv])
                x = (plsc.bitcast(lax.shift_left(ps, 16), jnp.float32)
                     + plsc.bitcast(
                         lax.bitwise_and(pt, jnp.int32(-65536)),
                         jnp.float32))
                x = jnp.maximum(x, 0.2 * x)
                x = jnp.minimum(jnp.maximum(x, -2.0), 2.0)
                att_b[bw, pl.ds(k * L, L)] = jnp.exp(x)
                return c3
            lax.fori_loop(0, C2 // L, att_k, 0)
            d_sa = pltpu.async_copy(
                att_b.at[bw], as_sh.at[dst8.at[cur, j]], sem_a.at[bw],
                add=True)

            d_g[b].wait()

            def wbody(j16, c3, b=b, bw=bw):
                attv = att_b[bw, pl.ds(j16 * L, L)]
                for i16 in range(L):
                    a = attv[i16]
                    i = j16 * L + i16
                    for kk in range(4):
                        x = hr[b, i, pl.ds(kk * L, L)]
                        ev = plsc.bitcast(lax.shift_left(x, 16),
                                          jnp.float32)
                        od = plsc.bitcast(
                            lax.bitwise_and(x, jnp.int32(-65536)),
                            jnp.float32)
                        wrows[bw, i, pl.ds(kk * 32, L)] = ev * a
                        wrows[bw, i, pl.ds(kk * 32 + L, L)] = od * a
                return c3
            lax.fori_loop(0, C2 // L, wbody, 0)
            d_su = pltpu.async_copy(
                wrows.at[bw], u_sh.at[dst8.at[cur, j]], sem_u.at[bw],
                add=True)
            pend[bw] = (d_su, d_sa)
        for b in range(2):
            if pend[b] is not None:
                pend[b][0].wait()
                pend[b][1].wait()
        d_ed.wait()
        d_es.wait()
        return c_
    lax.fori_loop(0, nblk, body, 0)

    plsc.subcore_barrier()

    pltpu.sync_copy(u_sh.at[pl.ds(sid * rows, rows)],
                    u_out.at[cid, pl.ds(sid * rows, rows)])
    pltpu.sync_copy(as_sh.at[pl.ds(sid * rows, rows)],
                    a_out.at[cid, pl.ds(sid * rows, rows)])


def kernel(node_states, edges, kernel, kernel_attention):
    n, d = node_states.shape
    u = kernel.shape[1]
    e = edges.shape[0]

    edges = edges.astype(jnp.int32)
    dst = edges[:, 0]
    src = edges[:, 1]

    rb = 512
    np_ = ((n + 1 + rb - 1) // rb) * rb
    kj = ((-(-e // (NW * C2)) + BLK - 1) // BLK) * BLK
    ep = NW * kj * C2
    kjs = kj
    kjf = kj

    ns_p = jnp.pad(node_states, ((0, np_ - n), (0, 0)))
    perm = []
    for blk32 in range(0, u, 32):
        for m in range(16):
            perm.extend([blk32 + m, blk32 + 16 + m])
    perm = jnp.array(perm, jnp.int32)
    at = kernel_attention.reshape(2, u)[:, perm]
    kernel = kernel[:, perm]
    dst_p = jnp.concatenate(
        [dst, jnp.full((ep - e,), n, jnp.int32)]).reshape(NW * kj, C2)
    src_p = jnp.concatenate(
        [src, jnp.zeros((ep - e,), jnp.int32)]).reshape(NW * kj, C2)

    h, s, t = pl.pallas_call(
        _tc_prep,
        grid=(np_ // rb,),
        in_specs=[
            pl.BlockSpec((rb, d), lambda i: (i, 0)),
            pl.BlockSpec((d, u), lambda i: (0, 0)),
            pl.BlockSpec((2, u), lambda i: (0, 0)),
        ],
        out_specs=[
            pl.BlockSpec((rb, u), lambda i: (i, 0)),
            pl.BlockSpec((rb,), lambda i: (i,)),
            pl.BlockSpec((rb,), lambda i: (i,)),
        ],
        out_shape=[
            jax.ShapeDtypeStruct((np_, u), jnp.bfloat16),
            jax.ShapeDtypeStruct((np_,), jnp.float32),
            jax.ShapeDtypeStruct((np_,), jnp.float32),
        ],
    )(ns_p, kernel, at)
    h = lax.bitcast_convert_type(h.reshape(np_, u // 2, 2), jnp.int32)
    st_pk = lax.bitcast_convert_type(
        jnp.stack([s.astype(jnp.bfloat16), t.astype(jnp.bfloat16)], -1),
        jnp.int32)

    mesh = plsc.VectorSubcoreMesh(core_axis_name="c", subcore_axis_name="s")
    u_part, a_part = pl.kernel(
        functools.partial(_sc_body, kj_split=(kjs, kjf), np_=np_),
        out_type=[
            jax.ShapeDtypeStruct((NC, np_, u), jnp.float32),
            jax.ShapeDtypeStruct((NC, np_), jnp.float32),
        ],
        mesh=mesh,
        compiler_params=pltpu.CompilerParams(needs_layout_passes=False, use_tc_tiling_on_sc=False),
        scratch_types=[
            pltpu.VMEM((2, BLK, C2), jnp.int32),
            pltpu.VMEM((2, BLK, C2), jnp.int32),
            pltpu.VMEM((2, C2), jnp.float32),
            pltpu.VMEM((np_,), jnp.int32),
            pltpu.VMEM((DEP, C2, u // 2), jnp.int32),
            pltpu.VMEM((2, C2, u), jnp.float32),
            pltpu.SemaphoreType.DMA((DEP,)),
            pltpu.SemaphoreType.DMA((2,)),
            pltpu.SemaphoreType.DMA((2,)),
            pltpu.SemaphoreType.DMA((2,)),
            pltpu.VMEM_SHARED((np_, u), jnp.float32),
            pltpu.VMEM_SHARED((np_,), jnp.float32),
        ],
    )(h, st_pk, dst_p, src_p)

    out = pl.pallas_call(
        _tc_finish,
        grid=(np_ // rb,),
        in_specs=[
            pl.BlockSpec((NC, rb, u), lambda i: (0, i, 0)),
            pl.BlockSpec((NC, rb), lambda i: (0, i)),
        ],
        out_specs=pl.BlockSpec((rb, u), lambda i: (i, 0)),
        out_shape=jax.ShapeDtypeStruct((np_, u), jnp.float32),
    )(u_part, a_part)

    return out[:n]
